# baseline (speedup 1.0000x reference)
import functools

import jax
import jax.numpy as jnp
from jax import lax
from jax.experimental import pallas as pl
from jax.experimental.pallas import tpu as pltpu
from jax.experimental.pallas import tpu_sc as plsc

VOCAB = 1002
EMBED = 64
BATCH = 4096
SEQ = 200
NUM_CORES = 2
NUM_SUBCORES = 16
NUM_WORKERS = NUM_CORES * NUM_SUBCORES
LANES = 16

BSLICE = 256
UNITS = SEQ * (BATCH // BSLICE)
UNITS_PER_W = UNITS // NUM_WORKERS
GROUPS = BSLICE // LANES
UNIT_WORDS = EMBED * BSLICE
EBLK_WORDS = 8 * 128
T_WORDS = EMBED * BATCH
EBLK_ROW_WORDS = 8 * BATCH
OUT_WORDS = BATCH * SEQ * EMBED
TAB_WORDS = VOCAB * EMBED

_mesh = plsc.VectorSubcoreMesh(core_axis_name="c", subcore_axis_name="s")


@functools.partial(
    pl.kernel,
    mesh=_mesh,
    out_type=jax.ShapeDtypeStruct((OUT_WORDS,), jnp.float32),
    scratch_types=[
        pltpu.VMEM((TAB_WORDS,), jnp.float32),
        pltpu.VMEM((2, UNIT_WORDS), jnp.float32),
        pltpu.VMEM((2, BSLICE), jnp.int32),
        pltpu.SemaphoreType.DMA,
    ],
    compiler_params=pltpu.CompilerParams(
        use_tc_tiling_on_sc=False, needs_layout_passes=False
    ),
)
def _embed_lookup(idx_hbm, w_hbm, out_hbm, tab_v, outbuf, idx_v, osem):
    wid = lax.axis_index("s") * NUM_CORES + lax.axis_index("c")
    u0 = wid * UNITS_PER_W

    pltpu.sync_copy(w_hbm, tab_v)

    def drain_out(par):
        pltpu.make_async_copy(
            outbuf.at[par],
            out_hbm.at[pl.ds(0, UNIT_WORDS)],
            osem,
        ).wait()

    def body(i, carry):
        u = u0 + i
        par = i % 2
        t = u // (BATCH // BSLICE)
        bq = u % (BATCH // BSLICE)
        b0 = bq * BSLICE

        pltpu.sync_copy(idx_hbm.at[pl.ds(t * BATCH + b0, BSLICE)], idx_v.at[par])

        @pl.when(i >= 2)
        def _():
            drain_out(par)

        def group(g, c2):
            idxv = idx_v[par, pl.ds(g * LANES, LANES)]
            rowbase = idxv * EMBED
            goff = g * LANES
            dyn = (goff // 128) * 1024 + goff % 128
            for e in range(EMBED):
                vals = plsc.load_gather(tab_v, [rowbase + e])
                off = (e // 8) * 2048 + (e % 8) * 128
                outbuf[par, pl.ds(dyn + off, LANES)] = vals
            return c2

        lax.fori_loop(0, GROUPS, group, 0)

        base = t * T_WORDS + b0 * 8
        for eb in range(8):
            pltpu.make_async_copy(
                outbuf.at[par, pl.ds(eb * 2048, 2048)],
                out_hbm.at[pl.ds(base + eb * EBLK_ROW_WORDS, 2048)],
                osem,
            ).start()
        return carry

    lax.fori_loop(0, UNITS_PER_W, body, 0)
    drain_out(0)
    drain_out(1)


def kernel(indices, W):
    idx_t = indices.T.reshape(-1).astype(jnp.int32)
    w_flat = W.reshape(-1)
    out = _embed_lookup(idx_t, w_flat)
    out5 = out.reshape(SEQ, 8, BATCH // 128, 8, 128)
    return out5.transpose(2, 4, 0, 1, 3).reshape(BATCH, SEQ, EMBED)


# speedup vs baseline: 9.5942x; 9.5942x over previous
import functools

import jax
import jax.numpy as jnp
from jax import lax
from jax.experimental import pallas as pl
from jax.experimental.pallas import tpu as pltpu
from jax.experimental.pallas import tpu_sc as plsc

VOCAB = 1002
EMBED = 64
BATCH = 4096
SEQ = 200
NUM_CORES = 2
NUM_SUBCORES = 16
NUM_WORKERS = NUM_CORES * NUM_SUBCORES
LANES = 16

BSLICE = 256
UNITS = SEQ * (BATCH // BSLICE)
UNITS_PER_W = UNITS // NUM_WORKERS
GROUPS = BSLICE // LANES
UNIT_WORDS = EMBED * BSLICE
EBLK_WORDS = 8 * 128
T_WORDS = EMBED * BATCH
EBLK_ROW_WORDS = 8 * BATCH
OUT_WORDS = BATCH * SEQ * EMBED
ROW_PAD = 65
TAB_WORDS = VOCAB * ROW_PAD
IDX_PER_W = UNITS_PER_W * BSLICE

_mesh = plsc.VectorSubcoreMesh(core_axis_name="c", subcore_axis_name="s")


@functools.partial(
    pl.kernel,
    mesh=_mesh,
    out_type=jax.ShapeDtypeStruct((OUT_WORDS,), jnp.float32),
    scratch_types=[
        pltpu.VMEM((TAB_WORDS,), jnp.float32),
        pltpu.VMEM((2, UNIT_WORDS), jnp.float32),
        pltpu.VMEM((IDX_PER_W,), jnp.int32),
        pltpu.SemaphoreType.DMA,
    ],
    compiler_params=pltpu.CompilerParams(
        use_tc_tiling_on_sc=False, needs_layout_passes=False
    ),
)
def _embed_lookup(idx_hbm, w_hbm, out_hbm, tab_v, outbuf, idx_v, osem):
    wid = lax.axis_index("s") * NUM_CORES + lax.axis_index("c")
    u0 = wid * UNITS_PER_W

    pltpu.sync_copy(w_hbm, tab_v)
    pltpu.sync_copy(idx_hbm.at[pl.ds(wid * IDX_PER_W, IDX_PER_W)], idx_v)

    def drain_out(par):
        pltpu.make_async_copy(
            outbuf.at[par],
            out_hbm.at[pl.ds(0, UNIT_WORDS)],
            osem,
        ).wait()

    def body(i, carry):
        u = u0 + i
        par = i % 2
        t = u // (BATCH // BSLICE)
        bq = u % (BATCH // BSLICE)
        b0 = bq * BSLICE

        @pl.when(i >= 2)
        def _():
            drain_out(par)

        @plsc.parallel_loop(0, GROUPS, 1, unroll=1)
        def group(g):
            idxv = idx_v[pl.ds(i * BSLICE + g * LANES, LANES)]
            rowbase = idxv * ROW_PAD
            goff = g * LANES
            dyn = (goff // 128) * 1024 + goff % 128
            for e in range(EMBED):
                vals = plsc.load_gather(tab_v, [rowbase + e])
                off = (e // 8) * 2048 + (e % 8) * 128
                outbuf[par, pl.ds(dyn + off, LANES)] = vals

        base = t * T_WORDS + b0 * 8
        for eb in range(8):
            pltpu.make_async_copy(
                outbuf.at[par, pl.ds(eb * 2048, 2048)],
                out_hbm.at[pl.ds(base + eb * EBLK_ROW_WORDS, 2048)],
                osem,
            ).start()
        return carry

    lax.fori_loop(0, UNITS_PER_W, body, 0)
    drain_out(0)
    drain_out(1)


def kernel(indices, W):
    idx_t = indices.T.reshape(-1).astype(jnp.int32)
    w_flat = jnp.pad(W, ((0, 0), (0, ROW_PAD - EMBED))).reshape(-1)
    out = _embed_lookup(idx_t, w_flat)
    out5 = out.reshape(SEQ, 8, BATCH // 128, 8, 128)
    return out5.transpose(2, 4, 0, 1, 3).reshape(BATCH, SEQ, EMBED)


# speedup vs baseline: 9.8552x; 1.0272x over previous
import functools

import jax
import jax.numpy as jnp
from jax import lax
from jax.experimental import pallas as pl
from jax.experimental.pallas import tpu as pltpu
from jax.experimental.pallas import tpu_sc as plsc

VOCAB = 1002
EMBED = 64
BATCH = 4096
SEQ = 200
NUM_CORES = 2
NUM_SUBCORES = 16
NUM_WORKERS = NUM_CORES * NUM_SUBCORES
LANES = 16

BSLICE = 256
UNITS = SEQ * (BATCH // BSLICE)
UNITS_PER_W = UNITS // NUM_WORKERS
GROUPS = BSLICE // LANES
UNIT_WORDS = EMBED * BSLICE
EBLK_WORDS = 8 * 128
T_WORDS = EMBED * BATCH
EBLK_ROW_WORDS = 8 * BATCH
OUT_WORDS = BATCH * SEQ * EMBED
ROW_PAD = 65
TAB_WORDS = VOCAB * ROW_PAD
IDX_PER_W = UNITS_PER_W * BSLICE

_mesh = plsc.VectorSubcoreMesh(core_axis_name="c", subcore_axis_name="s")


@functools.partial(
    pl.kernel,
    mesh=_mesh,
    out_type=jax.ShapeDtypeStruct((OUT_WORDS,), jnp.float32),
    scratch_types=[
        pltpu.VMEM((TAB_WORDS,), jnp.float32),
        pltpu.VMEM((2, UNIT_WORDS), jnp.float32),
        pltpu.VMEM((2, 2, 128), jnp.int32),
        pltpu.SemaphoreType.DMA,
        pltpu.SemaphoreType.DMA,
    ],
    compiler_params=pltpu.CompilerParams(
        use_tc_tiling_on_sc=False, needs_layout_passes=False
    ),
)
def _embed_lookup(idx_hbm, w_hbm, out_hbm, tab_v, outbuf, idx_v, osem, isem):
    wid = lax.axis_index("s") * NUM_CORES + lax.axis_index("c")
    u0 = wid * UNITS_PER_W

    def idx_src(u):
        return idx_hbm.at[u // 128, pl.ds((u % 16) * 2, 2), (u // 16) % 8]

    pltpu.sync_copy(w_hbm, tab_v)
    pltpu.sync_copy(idx_src(u0), idx_v.at[0])
    pltpu.make_async_copy(idx_src(u0 + 1), idx_v.at[1], isem).start()

    def drain_out(par):
        pltpu.make_async_copy(
            outbuf.at[par],
            out_hbm.at[pl.ds(0, UNIT_WORDS)],
            osem,
        ).wait()

    def body(i, carry):
        u = u0 + i
        par = i % 2
        t = u // (BATCH // BSLICE)
        bq = u % (BATCH // BSLICE)
        b0 = bq * BSLICE

        @pl.when(i >= 1)
        def _():
            pltpu.make_async_copy(idx_src(u), idx_v.at[par], isem).wait()

        @pl.when(i >= 2)
        def _():
            drain_out(par)

        @plsc.parallel_loop(0, GROUPS, 1, unroll=1)
        def group(g):
            goff = g * LANES
            idxv = idx_v[par, goff // 128, pl.ds(goff % 128, LANES)]
            rowbase = idxv * ROW_PAD
            dyn = (goff // 128) * 1024 + goff % 128
            for e in range(EMBED):
                vals = plsc.load_gather(tab_v, [rowbase + e])
                off = (e // 8) * 2048 + (e % 8) * 128
                outbuf[par, pl.ds(dyn + off, LANES)] = vals

        @pl.when(i + 2 < UNITS_PER_W)
        def _():
            pltpu.make_async_copy(idx_src(u + 2), idx_v.at[par], isem).start()

        base = t * T_WORDS + b0 * 8
        for eb in range(8):
            pltpu.make_async_copy(
                outbuf.at[par, pl.ds(eb * 2048, 2048)],
                out_hbm.at[pl.ds(base + eb * EBLK_ROW_WORDS, 2048)],
                osem,
            ).start()
        return carry

    lax.fori_loop(0, UNITS_PER_W, body, 0)
    drain_out(0)
    drain_out(1)


def kernel(indices, W):
    idx4 = (
        indices.astype(jnp.int32)
        .T.reshape(SEQ // 8, 8, BATCH // 128, 128)
        .transpose(0, 2, 1, 3)
    )
    w_flat = jnp.pad(W, ((0, 0), (0, ROW_PAD - EMBED))).reshape(-1)
    out = _embed_lookup(idx4, w_flat)
    out5 = out.reshape(SEQ, 8, BATCH // 128, 8, 128)
    return out5.transpose(2, 4, 0, 1, 3).reshape(BATCH, SEQ, EMBED)


# speedup vs baseline: 9.8823x; 1.0028x over previous
import functools

import jax
import jax.numpy as jnp
from jax import lax
from jax.experimental import pallas as pl
from jax.experimental.pallas import tpu as pltpu
from jax.experimental.pallas import tpu_sc as plsc

VOCAB = 1002
EMBED = 64
BATCH = 4096
SEQ = 200
NUM_CORES = 2
NUM_SUBCORES = 16
NUM_WORKERS = NUM_CORES * NUM_SUBCORES
LANES = 16

BSLICE = 256
UNITS = SEQ * (BATCH // BSLICE)
UNITS_PER_W = UNITS // NUM_WORKERS
GROUPS = BSLICE // LANES
UNIT_WORDS = EMBED * BSLICE
T_WORDS = EMBED * BATCH
EBLK_ROW_WORDS = 8 * BATCH
OUT_WORDS = BATCH * SEQ * EMBED
ROW_PAD = 65
TAB_WORDS = VOCAB * ROW_PAD

_mesh = plsc.VectorSubcoreMesh(core_axis_name="c", subcore_axis_name="s")


@functools.partial(
    pl.kernel,
    mesh=_mesh,
    out_type=jax.ShapeDtypeStruct((OUT_WORDS,), jnp.float32),
    scratch_types=[
        pltpu.VMEM((TAB_WORDS,), jnp.float32),
        pltpu.VMEM((2, UNIT_WORDS), jnp.float32),
        pltpu.VMEM((2, 2, 128), jnp.int32),
        pltpu.SemaphoreType.DMA,
        pltpu.SemaphoreType.DMA,
    ],
    compiler_params=pltpu.CompilerParams(
        use_tc_tiling_on_sc=False, needs_layout_passes=False
    ),
)
def _embed_lookup(idx_hbm, w_hbm, out_hbm, tab_v, outbuf, idx_v, osem, isem):
    wid = lax.axis_index("s") * NUM_CORES + lax.axis_index("c")
    u0 = wid * UNITS_PER_W

    def idx_src(u):
        return idx_hbm.at[u // 128, pl.ds((u % 16) * 2, 2), (u // 16) % 8]

    pltpu.sync_copy(w_hbm, tab_v)
    pltpu.sync_copy(idx_src(u0), idx_v.at[0])
    pltpu.make_async_copy(idx_src(u0 + 1), idx_v.at[1], isem).start()

    def drain_out(par):
        pltpu.make_async_copy(
            outbuf.at[par],
            out_hbm.at[pl.ds(0, UNIT_WORDS)],
            osem,
        ).wait()

    def body(i, carry):
        u = u0 + i
        par = i % 2
        t = u // (BATCH // BSLICE)
        bq = u % (BATCH // BSLICE)
        b0 = bq * BSLICE

        @pl.when(i >= 1)
        def _():
            pltpu.make_async_copy(idx_src(u), idx_v.at[par], isem).wait()

        @pl.when(i >= 2)
        def _():
            drain_out(par)

        @plsc.parallel_loop(0, GROUPS, 1, unroll=1)
        def group(g):
            goff = g * LANES
            idxv = idx_v[par, goff // 128, pl.ds(goff % 128, LANES)]
            rowbase = idxv * ROW_PAD
            dyn = (goff // 128) * 1024 + goff % 128
            for e in range(EMBED):
                vals = plsc.load_gather(tab_v, [rowbase + e])
                off = (e // 8) * 2048 + (e % 8) * 128
                outbuf[par, pl.ds(dyn + off, LANES)] = vals

        @pl.when(i + 2 < UNITS_PER_W)
        def _():
            pltpu.make_async_copy(idx_src(u + 2), idx_v.at[par], isem).start()

        base = t * T_WORDS + b0 * 8
        for eb in range(8):
            pltpu.make_async_copy(
                outbuf.at[par, pl.ds(eb * 2048, 2048)],
                out_hbm.at[pl.ds(base + eb * EBLK_ROW_WORDS, 2048)],
                osem,
            ).start()
        return carry

    lax.fori_loop(0, UNITS_PER_W, body, 0)
    drain_out(0)
    drain_out(1)


def kernel(indices, W):
    idx4 = (
        indices.astype(jnp.int32)
        .T.reshape(SEQ // 8, 8, BATCH // 128, 128)
        .transpose(0, 2, 1, 3)
    )
    w_flat = jnp.pad(W, ((0, 0), (0, ROW_PAD - EMBED))).reshape(-1)
    out = _embed_lookup(idx4, w_flat)
    out5 = out.reshape(SEQ, 8, BATCH // 128, 8, 128)
    return out5.transpose(2, 4, 0, 1, 3).reshape(BATCH, SEQ, EMBED)
